# baseline (speedup 1.0000x reference)
import functools

import jax
import jax.numpy as jnp
from jax import lax
from jax.experimental import pallas as pl
from jax.experimental.pallas import tpu as pltpu
from jax.experimental.pallas import tpu_sc as plsc

N = 10000
NPAD = 10112
D_IN = 128
D_HID = 128
D_C = 64
E = 320000
NC, NS = 2, 16
NT = NC * NS
EPT = E // NT
CH = 80
B = 128
EPT_PAD = CH * B
NBUF = 2
G = 16
RPT = NPAD // NS

_mesh = lambda: plsc.VectorSubcoreMesh(
    core_axis_name="c", subcore_axis_name="s", num_cores=NC, num_subcores=NS)


def _fill_f32(ref, value):
    if len(ref.shape) == 1:
        def body(i, _):
            ref[pl.ds(i * 16, 16)] = jnp.full((16,), value, jnp.float32)
            return 0
        lax.fori_loop(0, ref.shape[0] // 16, body, 0)
    else:
        rows, cols = ref.shape

        def body(i, _):
            r = i // (cols // 16)
            t = i % (cols // 16)
            ref[r, pl.ds(t * 16, 16)] = jnp.full((16,), value, jnp.float32)
            return 0
        lax.fori_loop(0, rows * (cols // 16), body, 0)


def _deg_call(dstp):
    @functools.partial(
        pl.kernel,
        out_type=jax.ShapeDtypeStruct((NC * NPAD,), jnp.float32),
        mesh=_mesh(),
        scratch_types=[
            pltpu.VMEM((CH, B), jnp.int32),
            pltpu.VMEM((640,), jnp.float32),
            pltpu.VMEM((B,), jnp.float32),
            pltpu.VMEM_SHARED((NPAD,), jnp.float32),
        ],
    )
    def deg_kernel(dstp_hbm, out_hbm, didx, zbuf, ones, acc):
        c = lax.axis_index("c")
        s = lax.axis_index("s")
        wid = c * NS + s
        _fill_f32(zbuf, 0.0)
        _fill_f32(ones, 1.0)
        pltpu.sync_copy(zbuf.at[pl.ds(0, RPT)], acc.at[pl.ds(s * RPT, RPT)])
        pltpu.sync_copy(dstp_hbm.at[wid], didx)
        plsc.subcore_barrier()

        def body(j, _):
            pltpu.sync_copy(ones, acc.at[didx.at[j]], add=True)
            return 0
        lax.fori_loop(0, CH, body, 0)
        plsc.subcore_barrier()
        pltpu.sync_copy(acc.at[pl.ds(s * RPT, RPT)], zbuf.at[pl.ds(0, RPT)])
        pltpu.sync_copy(zbuf.at[pl.ds(0, RPT)],
                        out_hbm.at[pl.ds(c * NPAD + s * RPT, RPT)])

    return deg_kernel(dstp)


def _propagate_call(hs, srcp, dstp, d):
    @functools.partial(
        pl.kernel,
        out_type=jax.ShapeDtypeStruct((NC, NPAD, d), jnp.float32),
        mesh=_mesh(),
        scratch_types=[
            pltpu.VMEM((CH, B), jnp.int32),
            pltpu.VMEM((CH, B), jnp.int32),
            pltpu.VMEM((B, d), jnp.float32),
            pltpu.VMEM_SHARED((NPAD, d), jnp.float32),
            pltpu.SemaphoreType.DMA,
        ],
    )
    def prop_kernel(hs_hbm, srcp_hbm, dstp_hbm, out_hbm, sidx, didx, rows,
                    acc, sem):
        c = lax.axis_index("c")
        s = lax.axis_index("s")
        wid = c * NS + s
        _fill_f32(rows, 0.0)
        base = s * RPT
        for k in range(RPT // B):
            pltpu.sync_copy(rows, acc.at[pl.ds(base + k * B, B)])
        rem = RPT % B
        if rem:
            pltpu.sync_copy(rows.at[pl.ds(0, rem)],
                            acc.at[pl.ds(base + (RPT // B) * B, rem)])
        pltpu.sync_copy(srcp_hbm.at[wid], sidx)
        pltpu.sync_copy(dstp_hbm.at[wid], didx)
        plsc.subcore_barrier()

        def body(j, _):
            pltpu.async_copy(hs_hbm.at[sidx.at[j]], rows, sem).wait()
            pltpu.sync_copy(rows, acc.at[didx.at[j]], add=True)
            return 0
        lax.fori_loop(0, CH, body, 0)
        plsc.subcore_barrier()
        off = 0
        for sz in (B, B, B, B, RPT - 4 * B):
            pltpu.sync_copy(acc.at[pl.ds(base + off, sz)],
                            rows.at[pl.ds(0, sz)])
            pltpu.sync_copy(rows.at[pl.ds(0, sz)],
                            out_hbm.at[c, pl.ds(base + off, sz)])
            off += sz

    return prop_kernel(hs, srcp, dstp)


RB = 632
GRID = NPAD // RB


def _stage_b(x, d0, d1, w1):
    def body(x_ref, d0_ref, d1_ref, w_ref, hs_ref, dinv_ref):
        deg = d0_ref[...] + d1_ref[...] + 1.0
        dinv = lax.rsqrt(deg)
        h = jnp.dot(x_ref[...], w_ref[...], preferred_element_type=jnp.float32)
        hs_ref[...] = h * dinv
        dinv_ref[...] = dinv

    return pl.pallas_call(
        body,
        grid=(GRID,),
        in_specs=[
            pl.BlockSpec((RB, D_IN), lambda i: (i, 0)),
            pl.BlockSpec((RB, 1), lambda i: (i, 0)),
            pl.BlockSpec((RB, 1), lambda i: (i, 0)),
            pl.BlockSpec((D_IN, D_HID), lambda i: (0, 0)),
        ],
        out_specs=[
            pl.BlockSpec((RB, D_HID), lambda i: (i, 0)),
            pl.BlockSpec((RB, 1), lambda i: (i, 0)),
        ],
        out_shape=[
            jax.ShapeDtypeStruct((NPAD, D_HID), jnp.float32),
            jax.ShapeDtypeStruct((NPAD, 1), jnp.float32),
        ],
    )(x, d0, d1, w1)


def _stage_mid(s0, s1, hs, dinv, b, w, d_out):
    d_in = hs.shape[1]

    def body(s0_ref, s1_ref, hs_ref, dinv_ref, b_ref, w_ref, o_ref):
        dinv = dinv_ref[...]
        pre = (s0_ref[...] + s1_ref[...] + hs_ref[...]) * dinv + b_ref[...]
        h = jnp.maximum(pre, 0.0)
        o_ref[...] = jnp.dot(h, w_ref[...],
                             preferred_element_type=jnp.float32) * dinv

    return pl.pallas_call(
        body,
        grid=(GRID,),
        in_specs=[
            pl.BlockSpec((RB, d_in), lambda i: (i, 0)),
            pl.BlockSpec((RB, d_in), lambda i: (i, 0)),
            pl.BlockSpec((RB, d_in), lambda i: (i, 0)),
            pl.BlockSpec((RB, 1), lambda i: (i, 0)),
            pl.BlockSpec((1, d_in), lambda i: (0, 0)),
            pl.BlockSpec((d_in, d_out), lambda i: (0, 0)),
        ],
        out_specs=pl.BlockSpec((RB, d_out), lambda i: (i, 0)),
        out_shape=jax.ShapeDtypeStruct((NPAD, d_out), jnp.float32),
    )(s0, s1, hs, dinv, b, w)


def _stage_z(u0, u1, hc, dinv, bc, noise):
    def body(u0_ref, u1_ref, hc_ref, dinv_ref, bc_ref, n_ref, z_ref):
        g = (u0_ref[...] + u1_ref[...] + hc_ref[...]) * dinv_ref[...]
        g = jnp.maximum(g + bc_ref[...], 0.0)
        mean = g[:, :32]
        ls = g[:, 32:64]
        z_ref[...] = n_ref[...] * jnp.exp(ls) + mean

    return pl.pallas_call(
        body,
        grid=(GRID,),
        in_specs=[
            pl.BlockSpec((RB, D_HID), lambda i: (i, 0)),
            pl.BlockSpec((RB, D_HID), lambda i: (i, 0)),
            pl.BlockSpec((RB, D_HID), lambda i: (i, 0)),
            pl.BlockSpec((RB, 1), lambda i: (i, 0)),
            pl.BlockSpec((1, D_HID), lambda i: (0, 0)),
            pl.BlockSpec((RB, 32), lambda i: (i, 0)),
        ],
        out_specs=pl.BlockSpec((RB, 32), lambda i: (i, 0)),
        out_shape=jax.ShapeDtypeStruct((NPAD, 32), jnp.float32),
    )(u0, u1, hc, dinv, bc, noise)


MB, NB = 400, 2048


def _decoder(z):
    gm = N // MB
    gn = pl.cdiv(N, NB)

    def body(zr_ref, zc_ref, o_ref):
        p = lax.dot_general(zr_ref[...], zc_ref[...],
                            (((1,), (1,)), ((), ())),
                            preferred_element_type=jnp.float32)
        rid = lax.broadcasted_iota(jnp.int32, (MB, NB), 0) + pl.program_id(0) * MB
        cid = lax.broadcasted_iota(jnp.int32, (MB, NB), 1) + pl.program_id(1) * NB
        o_ref[...] = jnp.where(cid > rid, jax.nn.sigmoid(p), 0.0)

    return pl.pallas_call(
        body,
        grid=(gm, gn),
        in_specs=[
            pl.BlockSpec((MB, 32), lambda i, j: (i, 0)),
            pl.BlockSpec((NB, 32), lambda i, j: (j, 0)),
        ],
        out_specs=pl.BlockSpec((MB, NB), lambda i, j: (i, j)),
        out_shape=jax.ShapeDtypeStruct((N, N), jnp.float32),
    )(z, z)


def kernel(x, edge_index, W1, b1, W2, b2, Wm, bm, Ws, bs):
    pad_idx = jnp.full((NT, EPT_PAD - EPT), N, jnp.int32)
    srcp = jnp.concatenate(
        [edge_index[0].reshape(NT, EPT), pad_idx], axis=1).reshape(NT, CH, B)
    dstp = jnp.concatenate(
        [edge_index[1].reshape(NT, EPT), pad_idx], axis=1).reshape(NT, CH, B)
    xp = jnp.pad(x, ((0, NPAD - N), (0, 0)))
    wc = jnp.pad(jnp.concatenate([Wm, Ws], axis=1), ((0, 0), (0, D_HID - D_C)))
    bc = jnp.pad(jnp.concatenate([bm, bs]), (0, D_HID - D_C)).reshape(1, D_HID)
    noise = jax.random.normal(jax.random.key(42), (N, 32), dtype=jnp.float32)
    noise = jnp.pad(noise, ((0, NPAD - N), (0, 0)))

    deg = _deg_call(dstp).reshape(NC, NPAD)
    d0 = deg[0].reshape(NPAD, 1)
    d1 = deg[1].reshape(NPAD, 1)
    hs1, dinv = _stage_b(xp, d0, d1, W1)

    s = _propagate_call(hs1, srcp, dstp, D_HID)
    hs2 = _stage_mid(s[0], s[1], hs1, dinv, b1.reshape(1, -1), W2, D_HID)
    t = _propagate_call(hs2, srcp, dstp, D_HID)
    hc = _stage_mid(t[0], t[1], hs2, dinv, b2.reshape(1, -1), wc, D_HID)
    u = _propagate_call(hc, srcp, dstp, D_HID)
    z = _stage_z(u[0], u[1], hc, dinv, bc, noise)

    return _decoder(z[:N])


# speedup vs baseline: 1.9327x; 1.9327x over previous
import functools

import jax
import jax.numpy as jnp
from jax import lax
from jax.experimental import pallas as pl
from jax.experimental.pallas import tpu as pltpu
from jax.experimental.pallas import tpu_sc as plsc

N = 10000
NPAD = 10112
D_IN = 128
D_HID = 128
D_C = 64
E = 320000
NC, NS = 2, 16
NT = NC * NS
EPT = E // NT
CH = 80
B = 128
EPT_PAD = CH * B
NBUF = 2
G = 16
RPT = NPAD // NS

_mesh = lambda: plsc.VectorSubcoreMesh(
    core_axis_name="c", subcore_axis_name="s", num_cores=NC, num_subcores=NS)


def _fill_f32(ref, value):
    if len(ref.shape) == 1:
        def body(i, _):
            ref[pl.ds(i * 16, 16)] = jnp.full((16,), value, jnp.float32)
            return 0
        lax.fori_loop(0, ref.shape[0] // 16, body, 0)
    else:
        rows, cols = ref.shape

        def body(i, _):
            r = i // (cols // 16)
            t = i % (cols // 16)
            ref[r, pl.ds(t * 16, 16)] = jnp.full((16,), value, jnp.float32)
            return 0
        lax.fori_loop(0, rows * (cols // 16), body, 0)


def _deg_call(dstp):
    @functools.partial(
        pl.kernel,
        out_type=jax.ShapeDtypeStruct((NC * NPAD,), jnp.float32),
        mesh=_mesh(),
        scratch_types=[
            pltpu.VMEM((CH, B), jnp.int32),
            pltpu.VMEM((640,), jnp.float32),
            pltpu.VMEM((B,), jnp.float32),
            pltpu.VMEM_SHARED((NPAD,), jnp.float32),
        ],
    )
    def deg_kernel(dstp_hbm, out_hbm, didx, zbuf, ones, acc):
        c = lax.axis_index("c")
        s = lax.axis_index("s")
        wid = c * NS + s
        _fill_f32(zbuf, 0.0)
        _fill_f32(ones, 1.0)
        pltpu.sync_copy(zbuf.at[pl.ds(0, RPT)], acc.at[pl.ds(s * RPT, RPT)])
        pltpu.sync_copy(dstp_hbm.at[wid], didx)
        plsc.subcore_barrier()

        def body(j, _):
            pltpu.sync_copy(ones, acc.at[didx.at[j]], add=True)
            return 0
        lax.fori_loop(0, CH, body, 0)
        plsc.subcore_barrier()
        pltpu.sync_copy(acc.at[pl.ds(s * RPT, RPT)], zbuf.at[pl.ds(0, RPT)])
        pltpu.sync_copy(zbuf.at[pl.ds(0, RPT)],
                        out_hbm.at[pl.ds(c * NPAD + s * RPT, RPT)])

    return deg_kernel(dstp)


def _propagate_call(hs, srcp, dstp, d):
    @functools.partial(
        pl.kernel,
        out_type=jax.ShapeDtypeStruct((NC, NPAD, d), jnp.float32),
        mesh=_mesh(),
        scratch_types=[
            pltpu.VMEM((CH, B), jnp.int32),
            pltpu.VMEM((CH, B), jnp.int32),
            pltpu.VMEM((B, d), jnp.float32),
            pltpu.VMEM_SHARED((NPAD, d), jnp.float32),
            pltpu.SemaphoreType.DMA,
        ],
    )
    def prop_kernel(hs_hbm, srcp_hbm, dstp_hbm, out_hbm, sidx, didx, rows,
                    acc, sem):
        c = lax.axis_index("c")
        s = lax.axis_index("s")
        wid = c * NS + s
        _fill_f32(rows, 0.0)
        base = s * RPT
        for k in range(RPT // B):
            pltpu.sync_copy(rows, acc.at[pl.ds(base + k * B, B)])
        rem = RPT % B
        if rem:
            pltpu.sync_copy(rows.at[pl.ds(0, rem)],
                            acc.at[pl.ds(base + (RPT // B) * B, rem)])
        pltpu.sync_copy(srcp_hbm.at[wid], sidx)
        pltpu.sync_copy(dstp_hbm.at[wid], didx)
        plsc.subcore_barrier()

        def body(j, _):
            pltpu.async_copy(hs_hbm.at[sidx.at[j]], rows, sem).wait()
            pltpu.sync_copy(rows, acc.at[didx.at[j]], add=True)
            return 0
        lax.fori_loop(0, CH, body, 0)
        plsc.subcore_barrier()
        off = 0
        for sz in (B, B, B, B, RPT - 4 * B):
            pltpu.sync_copy(acc.at[pl.ds(base + off, sz)],
                            rows.at[pl.ds(0, sz)])
            pltpu.sync_copy(rows.at[pl.ds(0, sz)],
                            out_hbm.at[c, pl.ds(base + off, sz)])
            off += sz

    return prop_kernel(hs, srcp, dstp)


RB = 632
GRID = NPAD // RB


def _stage_b(x, d0, d1, w1):
    def body(x_ref, d0_ref, d1_ref, w_ref, hs_ref, dinv_ref):
        deg = d0_ref[...] + d1_ref[...] + 1.0
        dinv = lax.rsqrt(deg)
        h = jnp.dot(x_ref[...], w_ref[...], preferred_element_type=jnp.float32)
        hs_ref[...] = h * dinv
        dinv_ref[...] = dinv

    return pl.pallas_call(
        body,
        grid=(GRID,),
        in_specs=[
            pl.BlockSpec((RB, D_IN), lambda i: (i, 0)),
            pl.BlockSpec((RB, 1), lambda i: (i, 0)),
            pl.BlockSpec((RB, 1), lambda i: (i, 0)),
            pl.BlockSpec((D_IN, D_HID), lambda i: (0, 0)),
        ],
        out_specs=[
            pl.BlockSpec((RB, D_HID), lambda i: (i, 0)),
            pl.BlockSpec((RB, 1), lambda i: (i, 0)),
        ],
        out_shape=[
            jax.ShapeDtypeStruct((NPAD, D_HID), jnp.float32),
            jax.ShapeDtypeStruct((NPAD, 1), jnp.float32),
        ],
    )(x, d0, d1, w1)


def _stage_mid(s0, s1, hs, dinv, b, w, d_out):
    d_in = hs.shape[1]

    def body(s0_ref, s1_ref, hs_ref, dinv_ref, b_ref, w_ref, o_ref):
        dinv = dinv_ref[...]
        pre = (s0_ref[...] + s1_ref[...] + hs_ref[...]) * dinv + b_ref[...]
        h = jnp.maximum(pre, 0.0)
        o_ref[...] = jnp.dot(h, w_ref[...],
                             preferred_element_type=jnp.float32) * dinv

    return pl.pallas_call(
        body,
        grid=(GRID,),
        in_specs=[
            pl.BlockSpec((RB, d_in), lambda i: (i, 0)),
            pl.BlockSpec((RB, d_in), lambda i: (i, 0)),
            pl.BlockSpec((RB, d_in), lambda i: (i, 0)),
            pl.BlockSpec((RB, 1), lambda i: (i, 0)),
            pl.BlockSpec((1, d_in), lambda i: (0, 0)),
            pl.BlockSpec((d_in, d_out), lambda i: (0, 0)),
        ],
        out_specs=pl.BlockSpec((RB, d_out), lambda i: (i, 0)),
        out_shape=jax.ShapeDtypeStruct((NPAD, d_out), jnp.float32),
    )(s0, s1, hs, dinv, b, w)


def _stage_z(u0, u1, hc, dinv, bc, noise):
    def body(u0_ref, u1_ref, hc_ref, dinv_ref, bc_ref, n_ref, z_ref):
        g = (u0_ref[...] + u1_ref[...] + hc_ref[...]) * dinv_ref[...]
        g = jnp.maximum(g + bc_ref[...], 0.0)
        mean = g[:, :32]
        ls = g[:, 32:64]
        z_ref[...] = n_ref[...] * jnp.exp(ls) + mean

    return pl.pallas_call(
        body,
        grid=(GRID,),
        in_specs=[
            pl.BlockSpec((RB, D_HID), lambda i: (i, 0)),
            pl.BlockSpec((RB, D_HID), lambda i: (i, 0)),
            pl.BlockSpec((RB, D_HID), lambda i: (i, 0)),
            pl.BlockSpec((RB, 1), lambda i: (i, 0)),
            pl.BlockSpec((1, D_HID), lambda i: (0, 0)),
            pl.BlockSpec((RB, 32), lambda i: (i, 0)),
        ],
        out_specs=pl.BlockSpec((RB, 32), lambda i: (i, 0)),
        out_shape=jax.ShapeDtypeStruct((NPAD, 32), jnp.float32),
    )(u0, u1, hc, dinv, bc, noise)


MB, NB = 400, 2048


def _decoder(z):
    gm = N // MB
    gn = pl.cdiv(N, NB)

    def body(zr_ref, zc_ref, o_ref):
        p = lax.dot_general(zr_ref[...], zc_ref[...],
                            (((1,), (1,)), ((), ())),
                            preferred_element_type=jnp.float32)
        rid = lax.broadcasted_iota(jnp.int32, (MB, NB), 0) + pl.program_id(0) * MB
        cid = lax.broadcasted_iota(jnp.int32, (MB, NB), 1) + pl.program_id(1) * NB
        o_ref[...] = jnp.where(cid > rid, jax.nn.sigmoid(p), 0.0)

    return pl.pallas_call(
        body,
        grid=(gm, gn),
        in_specs=[
            pl.BlockSpec((MB, 32), lambda i, j: (i, 0)),
            pl.BlockSpec((NB, 32), lambda i, j: (j, 0)),
        ],
        out_specs=pl.BlockSpec((MB, NB), lambda i, j: (i, j)),
        out_shape=jax.ShapeDtypeStruct((N, N), jnp.float32),
    )(z, z)


def kernel(x, edge_index, W1, b1, W2, b2, Wm, bm, Ws, bs):
    npad_rows = NPAD - N
    pad_idx = (N + jnp.arange(NT * (EPT_PAD - EPT), dtype=jnp.int32)
               % npad_rows).reshape(NT, EPT_PAD - EPT)
    srcp = jnp.concatenate(
        [edge_index[0].reshape(NT, EPT), pad_idx], axis=1).reshape(NT, CH, B)
    dstp = jnp.concatenate(
        [edge_index[1].reshape(NT, EPT), pad_idx], axis=1).reshape(NT, CH, B)
    xp = jnp.pad(x, ((0, NPAD - N), (0, 0)))
    wc = jnp.pad(jnp.concatenate([Wm, Ws], axis=1), ((0, 0), (0, D_HID - D_C)))
    bc = jnp.pad(jnp.concatenate([bm, bs]), (0, D_HID - D_C)).reshape(1, D_HID)
    noise = jax.random.normal(jax.random.key(42), (N, 32), dtype=jnp.float32)
    noise = jnp.pad(noise, ((0, NPAD - N), (0, 0)))

    deg = _deg_call(dstp).reshape(NC, NPAD)
    d0 = deg[0].reshape(NPAD, 1)
    d1 = deg[1].reshape(NPAD, 1)
    hs1, dinv = _stage_b(xp, d0, d1, W1)

    s = _propagate_call(hs1, srcp, dstp, D_HID)
    hs2 = _stage_mid(s[0], s[1], hs1, dinv, b1.reshape(1, -1), W2, D_HID)
    t = _propagate_call(hs2, srcp, dstp, D_HID)
    hc = _stage_mid(t[0], t[1], hs2, dinv, b2.reshape(1, -1), wc, D_HID)
    u = _propagate_call(hc, srcp, dstp, D_HID)
    z = _stage_z(u[0], u[1], hc, dinv, bc, noise)

    return _decoder(z[:N])


# speedup vs baseline: 2.3924x; 1.2378x over previous
import functools

import jax
import jax.numpy as jnp
from jax import lax
from jax.experimental import pallas as pl
from jax.experimental.pallas import tpu as pltpu
from jax.experimental.pallas import tpu_sc as plsc

N = 10000
NPAD = 10112
D_IN = 128
D_HID = 128
D_C = 64
E = 320000
NC, NS = 2, 16
NT = NC * NS
EPT = E // NT
CH = 80
B = 128
EPT_PAD = CH * B
NBUF = 2
G = 16
RPT = NPAD // NS

_mesh = lambda: plsc.VectorSubcoreMesh(
    core_axis_name="c", subcore_axis_name="s", num_cores=NC, num_subcores=NS)


def _fill_f32(ref, value):
    if len(ref.shape) == 1:
        def body(i, _):
            ref[pl.ds(i * 16, 16)] = jnp.full((16,), value, jnp.float32)
            return 0
        lax.fori_loop(0, ref.shape[0] // 16, body, 0)
    else:
        rows, cols = ref.shape

        def body(i, _):
            r = i // (cols // 16)
            t = i % (cols // 16)
            ref[r, pl.ds(t * 16, 16)] = jnp.full((16,), value, jnp.float32)
            return 0
        lax.fori_loop(0, rows * (cols // 16), body, 0)


def _deg_call(dstp):
    @functools.partial(
        pl.kernel,
        out_type=jax.ShapeDtypeStruct((NC * NPAD,), jnp.float32),
        mesh=_mesh(),
        scratch_types=[
            pltpu.VMEM((CH, B), jnp.int32),
            pltpu.VMEM((640,), jnp.float32),
            pltpu.VMEM((B,), jnp.float32),
            pltpu.VMEM_SHARED((NPAD,), jnp.float32),
        ],
    )
    def deg_kernel(dstp_hbm, out_hbm, didx, zbuf, ones, acc):
        c = lax.axis_index("c")
        s = lax.axis_index("s")
        wid = c * NS + s
        _fill_f32(zbuf, 0.0)
        _fill_f32(ones, 1.0)
        pltpu.sync_copy(zbuf.at[pl.ds(0, RPT)], acc.at[pl.ds(s * RPT, RPT)])
        pltpu.sync_copy(dstp_hbm.at[wid], didx)
        plsc.subcore_barrier()

        def body(j, _):
            pltpu.sync_copy(ones, acc.at[didx.at[j]], add=True)
            return 0
        lax.fori_loop(0, CH, body, 0)
        plsc.subcore_barrier()
        pltpu.sync_copy(acc.at[pl.ds(s * RPT, RPT)], zbuf.at[pl.ds(0, RPT)])
        pltpu.sync_copy(zbuf.at[pl.ds(0, RPT)],
                        out_hbm.at[pl.ds(c * NPAD + s * RPT, RPT)])

    return deg_kernel(dstp)


def _propagate_call(hs, srcp, dstp, d):
    @functools.partial(
        pl.kernel,
        out_type=jax.ShapeDtypeStruct((NC, NPAD, d), jnp.float32),
        mesh=_mesh(),
        scratch_types=[
            pltpu.VMEM((G, B), jnp.int32),
            pltpu.VMEM((G, B), jnp.int32),
            [pltpu.VMEM((B, d), jnp.float32)] * NBUF,
            pltpu.VMEM_SHARED((NPAD, d), jnp.float32),
            [pltpu.SemaphoreType.DMA] * NBUF,
            [pltpu.SemaphoreType.DMA] * NBUF,
        ],
    )
    def prop_kernel(hs_hbm, srcp_hbm, dstp_hbm, out_hbm, sidx, didx, rows,
                    acc, sg, ss):
        c = lax.axis_index("c")
        s = lax.axis_index("s")
        wid = c * NS + s
        _fill_f32(rows[0], 0.0)
        base = s * RPT
        for k in range(RPT // B):
            pltpu.sync_copy(rows[0], acc.at[pl.ds(base + k * B, B)])
        rem = RPT % B
        if rem:
            pltpu.sync_copy(rows[0].at[pl.ds(0, rem)],
                            acc.at[pl.ds(base + (RPT // B) * B, rem)])
        plsc.subcore_barrier()

        def group(grp, _):
            g0 = pl.multiple_of(grp * G, G)
            pltpu.sync_copy(srcp_hbm.at[wid, pl.ds(g0, G)], sidx)
            pltpu.sync_copy(dstp_hbm.at[wid, pl.ds(g0, G)], didx)
            gd = [None] * G
            sd = [None] * G
            for j in range(G):
                b = j % NBUF
                if j >= NBUF:
                    sd[j - NBUF].wait()
                gd[j] = pltpu.async_copy(
                    hs_hbm.at[sidx.at[j]], rows[b], sg[b])
                if j >= 1:
                    bb = (j - 1) % NBUF
                    gd[j - 1].wait()
                    sd[j - 1] = pltpu.async_copy(
                        rows[bb], acc.at[didx.at[j - 1]], ss[bb], add=True)
            bb = (G - 1) % NBUF
            gd[G - 1].wait()
            sd[G - 1] = pltpu.async_copy(
                rows[bb], acc.at[didx.at[G - 1]], ss[bb], add=True)
            for j in range(G - NBUF, G):
                sd[j].wait()
            return 0
        lax.fori_loop(0, CH // G, group, 0)
        plsc.subcore_barrier()
        off = 0
        for i, sz in enumerate((B, B, B, B, RPT - 4 * B)):
            buf = rows[i % NBUF].at[pl.ds(0, sz)]
            pltpu.sync_copy(acc.at[pl.ds(base + off, sz)], buf)
            pltpu.sync_copy(buf, out_hbm.at[c, pl.ds(base + off, sz)])
            off += sz

    return prop_kernel(hs, srcp, dstp)


RB = 632
GRID = NPAD // RB


def _stage_b(x, d0, d1, w1):
    def body(x_ref, d0_ref, d1_ref, w_ref, hs_ref, dinv_ref):
        deg = d0_ref[...] + d1_ref[...] + 1.0
        dinv = lax.rsqrt(deg)
        h = jnp.dot(x_ref[...], w_ref[...], preferred_element_type=jnp.float32)
        hs_ref[...] = h * dinv
        dinv_ref[...] = dinv

    return pl.pallas_call(
        body,
        grid=(GRID,),
        in_specs=[
            pl.BlockSpec((RB, D_IN), lambda i: (i, 0)),
            pl.BlockSpec((RB, 1), lambda i: (i, 0)),
            pl.BlockSpec((RB, 1), lambda i: (i, 0)),
            pl.BlockSpec((D_IN, D_HID), lambda i: (0, 0)),
        ],
        out_specs=[
            pl.BlockSpec((RB, D_HID), lambda i: (i, 0)),
            pl.BlockSpec((RB, 1), lambda i: (i, 0)),
        ],
        out_shape=[
            jax.ShapeDtypeStruct((NPAD, D_HID), jnp.float32),
            jax.ShapeDtypeStruct((NPAD, 1), jnp.float32),
        ],
    )(x, d0, d1, w1)


def _stage_mid(s0, s1, hs, dinv, b, w, d_out):
    d_in = hs.shape[1]

    def body(s0_ref, s1_ref, hs_ref, dinv_ref, b_ref, w_ref, o_ref):
        dinv = dinv_ref[...]
        pre = (s0_ref[...] + s1_ref[...] + hs_ref[...]) * dinv + b_ref[...]
        h = jnp.maximum(pre, 0.0)
        o_ref[...] = jnp.dot(h, w_ref[...],
                             preferred_element_type=jnp.float32) * dinv

    return pl.pallas_call(
        body,
        grid=(GRID,),
        in_specs=[
            pl.BlockSpec((RB, d_in), lambda i: (i, 0)),
            pl.BlockSpec((RB, d_in), lambda i: (i, 0)),
            pl.BlockSpec((RB, d_in), lambda i: (i, 0)),
            pl.BlockSpec((RB, 1), lambda i: (i, 0)),
            pl.BlockSpec((1, d_in), lambda i: (0, 0)),
            pl.BlockSpec((d_in, d_out), lambda i: (0, 0)),
        ],
        out_specs=pl.BlockSpec((RB, d_out), lambda i: (i, 0)),
        out_shape=jax.ShapeDtypeStruct((NPAD, d_out), jnp.float32),
    )(s0, s1, hs, dinv, b, w)


def _stage_z(u0, u1, hc, dinv, bc, noise):
    def body(u0_ref, u1_ref, hc_ref, dinv_ref, bc_ref, n_ref, z_ref):
        g = (u0_ref[...] + u1_ref[...] + hc_ref[...]) * dinv_ref[...]
        g = jnp.maximum(g + bc_ref[...], 0.0)
        mean = g[:, :32]
        ls = g[:, 32:64]
        z_ref[...] = n_ref[...] * jnp.exp(ls) + mean

    return pl.pallas_call(
        body,
        grid=(GRID,),
        in_specs=[
            pl.BlockSpec((RB, D_HID), lambda i: (i, 0)),
            pl.BlockSpec((RB, D_HID), lambda i: (i, 0)),
            pl.BlockSpec((RB, D_HID), lambda i: (i, 0)),
            pl.BlockSpec((RB, 1), lambda i: (i, 0)),
            pl.BlockSpec((1, D_HID), lambda i: (0, 0)),
            pl.BlockSpec((RB, 32), lambda i: (i, 0)),
        ],
        out_specs=pl.BlockSpec((RB, 32), lambda i: (i, 0)),
        out_shape=jax.ShapeDtypeStruct((NPAD, 32), jnp.float32),
    )(u0, u1, hc, dinv, bc, noise)


MB, NB = 400, 2048


def _decoder(z):
    gm = N // MB
    gn = pl.cdiv(N, NB)

    def body(zr_ref, zc_ref, o_ref):
        p = lax.dot_general(zr_ref[...], zc_ref[...],
                            (((1,), (1,)), ((), ())),
                            preferred_element_type=jnp.float32)
        rid = lax.broadcasted_iota(jnp.int32, (MB, NB), 0) + pl.program_id(0) * MB
        cid = lax.broadcasted_iota(jnp.int32, (MB, NB), 1) + pl.program_id(1) * NB
        o_ref[...] = jnp.where(cid > rid, jax.nn.sigmoid(p), 0.0)

    return pl.pallas_call(
        body,
        grid=(gm, gn),
        in_specs=[
            pl.BlockSpec((MB, 32), lambda i, j: (i, 0)),
            pl.BlockSpec((NB, 32), lambda i, j: (j, 0)),
        ],
        out_specs=pl.BlockSpec((MB, NB), lambda i, j: (i, j)),
        out_shape=jax.ShapeDtypeStruct((N, N), jnp.float32),
    )(z, z)


def kernel(x, edge_index, W1, b1, W2, b2, Wm, bm, Ws, bs):
    npad_rows = NPAD - N
    pad_idx = (N + jnp.arange(NT * (EPT_PAD - EPT), dtype=jnp.int32)
               % npad_rows).reshape(NT, EPT_PAD - EPT)
    srcp = jnp.concatenate(
        [edge_index[0].reshape(NT, EPT), pad_idx], axis=1).reshape(NT, CH, B)
    dstp = jnp.concatenate(
        [edge_index[1].reshape(NT, EPT), pad_idx], axis=1).reshape(NT, CH, B)
    xp = jnp.pad(x, ((0, NPAD - N), (0, 0)))
    wc = jnp.pad(jnp.concatenate([Wm, Ws], axis=1), ((0, 0), (0, D_HID - D_C)))
    bc = jnp.pad(jnp.concatenate([bm, bs]), (0, D_HID - D_C)).reshape(1, D_HID)
    noise = jax.random.normal(jax.random.key(42), (N, 32), dtype=jnp.float32)
    noise = jnp.pad(noise, ((0, NPAD - N), (0, 0)))

    deg = _deg_call(dstp).reshape(NC, NPAD)
    d0 = deg[0].reshape(NPAD, 1)
    d1 = deg[1].reshape(NPAD, 1)
    hs1, dinv = _stage_b(xp, d0, d1, W1)

    s = _propagate_call(hs1, srcp, dstp, D_HID)
    hs2 = _stage_mid(s[0], s[1], hs1, dinv, b1.reshape(1, -1), W2, D_HID)
    t = _propagate_call(hs2, srcp, dstp, D_HID)
    hc = _stage_mid(t[0], t[1], hs2, dinv, b2.reshape(1, -1), wc, D_HID)
    u = _propagate_call(hc, srcp, dstp, D_HID)
    z = _stage_z(u[0], u[1], hc, dinv, bc, noise)

    return _decoder(z[:N])


# speedup vs baseline: 2.5946x; 1.0845x over previous
import functools

import jax
import jax.numpy as jnp
from jax import lax
from jax.experimental import pallas as pl
from jax.experimental.pallas import tpu as pltpu
from jax.experimental.pallas import tpu_sc as plsc

N = 10000
NPAD = 10112
D_IN = 128
D_HID = 128
D_C = 64
E = 320000
NC, NS = 2, 16
NT = NC * NS
EPT = E // NT
CH = 80
B = 128
EPT_PAD = CH * B
NBUF = 2
G = 16
RPT = NPAD // NS

_mesh = lambda: plsc.VectorSubcoreMesh(
    core_axis_name="c", subcore_axis_name="s", num_cores=NC, num_subcores=NS)


def _fill_f32(ref, value):
    if len(ref.shape) == 1:
        def body(i, _):
            ref[pl.ds(i * 16, 16)] = jnp.full((16,), value, jnp.float32)
            return 0
        lax.fori_loop(0, ref.shape[0] // 16, body, 0)
    else:
        rows, cols = ref.shape

        def body(i, _):
            r = i // (cols // 16)
            t = i % (cols // 16)
            ref[r, pl.ds(t * 16, 16)] = jnp.full((16,), value, jnp.float32)
            return 0
        lax.fori_loop(0, rows * (cols // 16), body, 0)


def _deg_call(dstp):
    @functools.partial(
        pl.kernel,
        out_type=jax.ShapeDtypeStruct((NC * NPAD,), jnp.float32),
        mesh=_mesh(),
        scratch_types=[
            pltpu.VMEM((CH, B), jnp.int32),
            pltpu.VMEM((640,), jnp.float32),
            pltpu.VMEM((B,), jnp.float32),
            pltpu.VMEM_SHARED((NPAD,), jnp.float32),
        ],
    )
    def deg_kernel(dstp_hbm, out_hbm, didx, zbuf, ones, acc):
        c = lax.axis_index("c")
        s = lax.axis_index("s")
        wid = c * NS + s
        _fill_f32(zbuf, 0.0)
        _fill_f32(ones, 1.0)
        pltpu.sync_copy(zbuf.at[pl.ds(0, RPT)], acc.at[pl.ds(s * RPT, RPT)])
        pltpu.sync_copy(dstp_hbm.at[wid], didx)
        plsc.subcore_barrier()

        def body(j, _):
            pltpu.sync_copy(ones, acc.at[didx.at[j]], add=True)
            return 0
        lax.fori_loop(0, CH, body, 0)
        plsc.subcore_barrier()
        pltpu.sync_copy(acc.at[pl.ds(s * RPT, RPT)], zbuf.at[pl.ds(0, RPT)])
        pltpu.sync_copy(zbuf.at[pl.ds(0, RPT)],
                        out_hbm.at[pl.ds(c * NPAD + s * RPT, RPT)])

    return deg_kernel(dstp)


def _propagate_call(hs, srcp, dstp, d):
    @functools.partial(
        pl.kernel,
        out_type=jax.ShapeDtypeStruct((NC, NPAD, d), jnp.float32),
        mesh=_mesh(),
        scratch_types=[
            pltpu.VMEM((G, B), jnp.int32),
            pltpu.VMEM((G, B), jnp.int32),
            [pltpu.VMEM((B, d), jnp.float32)] * NBUF,
            pltpu.VMEM_SHARED((NPAD, d), jnp.float32),
            [pltpu.SemaphoreType.DMA] * NBUF,
            [pltpu.SemaphoreType.DMA] * NBUF,
        ],
    )
    def prop_kernel(hs_hbm, srcp_hbm, dstp_hbm, out_hbm, sidx, didx, rows,
                    acc, sg, ss):
        c = lax.axis_index("c")
        s = lax.axis_index("s")
        wid = c * NS + s
        _fill_f32(rows[0], 0.0)
        base = s * RPT
        for k in range(RPT // B):
            pltpu.sync_copy(rows[0], acc.at[pl.ds(base + k * B, B)])
        rem = RPT % B
        if rem:
            pltpu.sync_copy(rows[0].at[pl.ds(0, rem)],
                            acc.at[pl.ds(base + (RPT // B) * B, rem)])
        plsc.subcore_barrier()

        def group(grp, _):
            g0 = pl.multiple_of(grp * G, G)
            pltpu.sync_copy(srcp_hbm.at[wid, pl.ds(g0, G)], sidx)
            pltpu.sync_copy(dstp_hbm.at[wid, pl.ds(g0, G)], didx)
            gd = [None] * G
            sd = [None] * G
            for j in range(G):
                b = j % NBUF
                if j >= NBUF:
                    sd[j - NBUF].wait()
                gd[j] = pltpu.async_copy(
                    hs_hbm.at[sidx.at[j]], rows[b], sg[b])
                if j >= 1:
                    bb = (j - 1) % NBUF
                    gd[j - 1].wait()
                    sd[j - 1] = pltpu.async_copy(
                        rows[bb], acc.at[didx.at[j - 1]], ss[bb], add=True)
            bb = (G - 1) % NBUF
            gd[G - 1].wait()
            sd[G - 1] = pltpu.async_copy(
                rows[bb], acc.at[didx.at[G - 1]], ss[bb], add=True)
            for j in range(G - NBUF, G):
                sd[j].wait()
            return 0
        lax.fori_loop(0, CH // G, group, 0)
        plsc.subcore_barrier()
        off = 0
        for i, sz in enumerate((B, B, B, B, RPT - 4 * B)):
            buf = rows[i % NBUF].at[pl.ds(0, sz)]
            pltpu.sync_copy(acc.at[pl.ds(base + off, sz)], buf)
            pltpu.sync_copy(buf, out_hbm.at[c, pl.ds(base + off, sz)])
            off += sz

    return prop_kernel(hs, srcp, dstp)


RB = 632
GRID = NPAD // RB


def _stage_b(x, d0, d1, w1):
    def body(x_ref, d0_ref, d1_ref, w_ref, hs_ref, dinv_ref):
        deg = d0_ref[...] + d1_ref[...] + 1.0
        dinv = lax.rsqrt(deg)
        h = jnp.dot(x_ref[...], w_ref[...], preferred_element_type=jnp.float32)
        hs_ref[...] = h * dinv
        dinv_ref[...] = dinv

    return pl.pallas_call(
        body,
        grid=(GRID,),
        in_specs=[
            pl.BlockSpec((RB, D_IN), lambda i: (i, 0)),
            pl.BlockSpec((RB, 1), lambda i: (i, 0)),
            pl.BlockSpec((RB, 1), lambda i: (i, 0)),
            pl.BlockSpec((D_IN, D_HID), lambda i: (0, 0)),
        ],
        out_specs=[
            pl.BlockSpec((RB, D_HID), lambda i: (i, 0)),
            pl.BlockSpec((RB, 1), lambda i: (i, 0)),
        ],
        out_shape=[
            jax.ShapeDtypeStruct((NPAD, D_HID), jnp.float32),
            jax.ShapeDtypeStruct((NPAD, 1), jnp.float32),
        ],
    )(x, d0, d1, w1)


def _stage_mid(s0, s1, hs, dinv, b, w, d_out):
    d_in = hs.shape[1]

    def body(s0_ref, s1_ref, hs_ref, dinv_ref, b_ref, w_ref, o_ref):
        dinv = dinv_ref[...]
        pre = (s0_ref[...] + s1_ref[...] + hs_ref[...]) * dinv + b_ref[...]
        h = jnp.maximum(pre, 0.0)
        o_ref[...] = jnp.dot(h, w_ref[...],
                             preferred_element_type=jnp.float32) * dinv

    return pl.pallas_call(
        body,
        grid=(GRID,),
        in_specs=[
            pl.BlockSpec((RB, d_in), lambda i: (i, 0)),
            pl.BlockSpec((RB, d_in), lambda i: (i, 0)),
            pl.BlockSpec((RB, d_in), lambda i: (i, 0)),
            pl.BlockSpec((RB, 1), lambda i: (i, 0)),
            pl.BlockSpec((1, d_in), lambda i: (0, 0)),
            pl.BlockSpec((d_in, d_out), lambda i: (0, 0)),
        ],
        out_specs=pl.BlockSpec((RB, d_out), lambda i: (i, 0)),
        out_shape=jax.ShapeDtypeStruct((NPAD, d_out), jnp.float32),
    )(s0, s1, hs, dinv, b, w)


def _stage_z(u0, u1, hc, dinv, bc, noise):
    def body(u0_ref, u1_ref, hc_ref, dinv_ref, bc_ref, n_ref, z_ref):
        g = (u0_ref[...] + u1_ref[...] + hc_ref[...]) * dinv_ref[...]
        g = jnp.maximum(g + bc_ref[...], 0.0)
        mean = g[:, :32]
        ls = g[:, 32:64]
        z_ref[...] = n_ref[...] * jnp.exp(ls) + mean

    return pl.pallas_call(
        body,
        grid=(GRID,),
        in_specs=[
            pl.BlockSpec((RB, D_HID), lambda i: (i, 0)),
            pl.BlockSpec((RB, D_HID), lambda i: (i, 0)),
            pl.BlockSpec((RB, D_HID), lambda i: (i, 0)),
            pl.BlockSpec((RB, 1), lambda i: (i, 0)),
            pl.BlockSpec((1, D_HID), lambda i: (0, 0)),
            pl.BlockSpec((RB, 32), lambda i: (i, 0)),
        ],
        out_specs=pl.BlockSpec((RB, 32), lambda i: (i, 0)),
        out_shape=jax.ShapeDtypeStruct((NPAD, 32), jnp.float32),
    )(u0, u1, hc, dinv, bc, noise)


MB, NB = 1000, 4096


def _decoder(z):
    gm = N // MB
    gn = pl.cdiv(N, NB)

    def body(zr_ref, zc_ref, o_ref):
        p = lax.dot_general(zr_ref[...], zc_ref[...],
                            (((1,), (1,)), ((), ())),
                            preferred_element_type=jnp.float32)
        rid = lax.broadcasted_iota(jnp.int32, (MB, NB), 0) + pl.program_id(0) * MB
        cid = lax.broadcasted_iota(jnp.int32, (MB, NB), 1) + pl.program_id(1) * NB
        o_ref[...] = jnp.where(cid > rid, jax.nn.sigmoid(p), 0.0)

    return pl.pallas_call(
        body,
        grid=(gm, gn),
        in_specs=[
            pl.BlockSpec((MB, 32), lambda i, j: (i, 0)),
            pl.BlockSpec((NB, 32), lambda i, j: (j, 0)),
        ],
        out_specs=pl.BlockSpec((MB, NB), lambda i, j: (i, j)),
        out_shape=jax.ShapeDtypeStruct((N, N), jnp.float32),
    )(z, z)


def kernel(x, edge_index, W1, b1, W2, b2, Wm, bm, Ws, bs):
    npad_rows = NPAD - N
    pad_idx = (N + jnp.arange(NT * (EPT_PAD - EPT), dtype=jnp.int32)
               % npad_rows).reshape(NT, EPT_PAD - EPT)
    srcp = jnp.concatenate(
        [edge_index[0].reshape(NT, EPT), pad_idx], axis=1).reshape(NT, CH, B)
    dstp = jnp.concatenate(
        [edge_index[1].reshape(NT, EPT), pad_idx], axis=1).reshape(NT, CH, B)
    xp = jnp.pad(x, ((0, NPAD - N), (0, 0)))
    wc = jnp.pad(jnp.concatenate([Wm, Ws], axis=1), ((0, 0), (0, D_HID - D_C)))
    bc = jnp.pad(jnp.concatenate([bm, bs]), (0, D_HID - D_C)).reshape(1, D_HID)
    noise = jax.random.normal(jax.random.key(42), (N, 32), dtype=jnp.float32)
    noise = jnp.pad(noise, ((0, NPAD - N), (0, 0)))

    deg = _deg_call(dstp).reshape(NC, NPAD)
    d0 = deg[0].reshape(NPAD, 1)
    d1 = deg[1].reshape(NPAD, 1)
    hs1, dinv = _stage_b(xp, d0, d1, W1)

    s = _propagate_call(hs1, srcp, dstp, D_HID)
    hs2 = _stage_mid(s[0], s[1], hs1, dinv, b1.reshape(1, -1), W2, D_HID)
    t = _propagate_call(hs2, srcp, dstp, D_HID)
    hc = _stage_mid(t[0], t[1], hs2, dinv, b2.reshape(1, -1), wc, D_HID)
    u = _propagate_call(hc, srcp, dstp, D_HID)
    z = _stage_z(u[0], u[1], hc, dinv, bc, noise)

    return _decoder(z[:N])


# speedup vs baseline: 2.6911x; 1.0372x over previous
import functools

import jax
import jax.numpy as jnp
from jax import lax
from jax.experimental import pallas as pl
from jax.experimental.pallas import tpu as pltpu
from jax.experimental.pallas import tpu_sc as plsc

N = 10000
NPAD = 10112
D_IN = 128
D_HID = 128
D_C = 64
E = 320000
NC, NS = 2, 16
NT = NC * NS
EPT = E // NT
CH = 80
B = 128
EPT_PAD = CH * B
NBUF = 2
G = 40
RPT = NPAD // NS

_mesh = lambda: plsc.VectorSubcoreMesh(
    core_axis_name="c", subcore_axis_name="s", num_cores=NC, num_subcores=NS)


def _fill_f32(ref, value):
    if len(ref.shape) == 1:
        def body(i, _):
            ref[pl.ds(i * 16, 16)] = jnp.full((16,), value, jnp.float32)
            return 0
        lax.fori_loop(0, ref.shape[0] // 16, body, 0)
    else:
        rows, cols = ref.shape

        def body(i, _):
            r = i // (cols // 16)
            t = i % (cols // 16)
            ref[r, pl.ds(t * 16, 16)] = jnp.full((16,), value, jnp.float32)
            return 0
        lax.fori_loop(0, rows * (cols // 16), body, 0)


def _deg_call(dstp):
    @functools.partial(
        pl.kernel,
        out_type=jax.ShapeDtypeStruct((NC * NPAD,), jnp.float32),
        mesh=_mesh(),
        scratch_types=[
            pltpu.VMEM((CH, B), jnp.int32),
            pltpu.VMEM((640,), jnp.float32),
            pltpu.VMEM((B,), jnp.float32),
            pltpu.VMEM_SHARED((NPAD,), jnp.float32),
        ],
    )
    def deg_kernel(dstp_hbm, out_hbm, didx, zbuf, ones, acc):
        c = lax.axis_index("c")
        s = lax.axis_index("s")
        wid = c * NS + s
        _fill_f32(zbuf, 0.0)
        _fill_f32(ones, 1.0)
        pltpu.sync_copy(zbuf.at[pl.ds(0, RPT)], acc.at[pl.ds(s * RPT, RPT)])
        pltpu.sync_copy(dstp_hbm.at[wid], didx)
        plsc.subcore_barrier()

        def body(j, _):
            pltpu.sync_copy(ones, acc.at[didx.at[j]], add=True)
            return 0
        lax.fori_loop(0, CH, body, 0)
        plsc.subcore_barrier()
        pltpu.sync_copy(acc.at[pl.ds(s * RPT, RPT)], zbuf.at[pl.ds(0, RPT)])
        pltpu.sync_copy(zbuf.at[pl.ds(0, RPT)],
                        out_hbm.at[pl.ds(c * NPAD + s * RPT, RPT)])

    return deg_kernel(dstp)


def _propagate_call(hs, srcp, dstp, d):
    @functools.partial(
        pl.kernel,
        out_type=jax.ShapeDtypeStruct((NC, NPAD, d), jnp.float32),
        mesh=_mesh(),
        scratch_types=[
            pltpu.VMEM((G, B), jnp.int32),
            pltpu.VMEM((G, B), jnp.int32),
            [pltpu.VMEM((B, d), jnp.float32)] * NBUF,
            pltpu.VMEM_SHARED((NPAD, d), jnp.float32),
            [pltpu.SemaphoreType.DMA] * NBUF,
            [pltpu.SemaphoreType.DMA] * NBUF,
        ],
    )
    def prop_kernel(hs_hbm, srcp_hbm, dstp_hbm, out_hbm, sidx, didx, rows,
                    acc, sg, ss):
        c = lax.axis_index("c")
        s = lax.axis_index("s")
        wid = c * NS + s
        _fill_f32(rows[0], 0.0)
        base = s * RPT
        for k in range(RPT // B):
            pltpu.sync_copy(rows[0], acc.at[pl.ds(base + k * B, B)])
        rem = RPT % B
        if rem:
            pltpu.sync_copy(rows[0].at[pl.ds(0, rem)],
                            acc.at[pl.ds(base + (RPT // B) * B, rem)])
        plsc.subcore_barrier()

        def group(grp, _):
            g0 = pl.multiple_of(grp * G, G)
            pltpu.sync_copy(srcp_hbm.at[wid, pl.ds(g0, G)], sidx)
            pltpu.sync_copy(dstp_hbm.at[wid, pl.ds(g0, G)], didx)
            gd = [None] * G
            sd = [None] * G
            for j in range(G):
                b = j % NBUF
                if j >= NBUF:
                    sd[j - NBUF].wait()
                gd[j] = pltpu.async_copy(
                    hs_hbm.at[sidx.at[j]], rows[b], sg[b])
                if j >= 1:
                    bb = (j - 1) % NBUF
                    gd[j - 1].wait()
                    sd[j - 1] = pltpu.async_copy(
                        rows[bb], acc.at[didx.at[j - 1]], ss[bb], add=True)
            bb = (G - 1) % NBUF
            gd[G - 1].wait()
            sd[G - 1] = pltpu.async_copy(
                rows[bb], acc.at[didx.at[G - 1]], ss[bb], add=True)
            for j in range(G - NBUF, G):
                sd[j].wait()
            return 0
        lax.fori_loop(0, CH // G, group, 0)
        plsc.subcore_barrier()
        off = 0
        for i, sz in enumerate((B, B, B, B, RPT - 4 * B)):
            buf = rows[i % NBUF].at[pl.ds(0, sz)]
            pltpu.sync_copy(acc.at[pl.ds(base + off, sz)], buf)
            pltpu.sync_copy(buf, out_hbm.at[c, pl.ds(base + off, sz)])
            off += sz

    return prop_kernel(hs, srcp, dstp)


RB = 632
GRID = NPAD // RB


def _stage_b(x, d0, d1, w1):
    def body(x_ref, d0_ref, d1_ref, w_ref, hs_ref, dinv_ref):
        deg = d0_ref[...] + d1_ref[...] + 1.0
        dinv = lax.rsqrt(deg)
        h = jnp.dot(x_ref[...], w_ref[...], preferred_element_type=jnp.float32)
        hs_ref[...] = h * dinv
        dinv_ref[...] = dinv

    return pl.pallas_call(
        body,
        grid=(GRID,),
        in_specs=[
            pl.BlockSpec((RB, D_IN), lambda i: (i, 0)),
            pl.BlockSpec((RB, 1), lambda i: (i, 0)),
            pl.BlockSpec((RB, 1), lambda i: (i, 0)),
            pl.BlockSpec((D_IN, D_HID), lambda i: (0, 0)),
        ],
        out_specs=[
            pl.BlockSpec((RB, D_HID), lambda i: (i, 0)),
            pl.BlockSpec((RB, 1), lambda i: (i, 0)),
        ],
        out_shape=[
            jax.ShapeDtypeStruct((NPAD, D_HID), jnp.float32),
            jax.ShapeDtypeStruct((NPAD, 1), jnp.float32),
        ],
    )(x, d0, d1, w1)


def _stage_mid(s0, s1, hs, dinv, b, w, d_out):
    d_in = hs.shape[1]

    def body(s0_ref, s1_ref, hs_ref, dinv_ref, b_ref, w_ref, o_ref):
        dinv = dinv_ref[...]
        pre = (s0_ref[...] + s1_ref[...] + hs_ref[...]) * dinv + b_ref[...]
        h = jnp.maximum(pre, 0.0)
        o_ref[...] = jnp.dot(h, w_ref[...],
                             preferred_element_type=jnp.float32) * dinv

    return pl.pallas_call(
        body,
        grid=(GRID,),
        in_specs=[
            pl.BlockSpec((RB, d_in), lambda i: (i, 0)),
            pl.BlockSpec((RB, d_in), lambda i: (i, 0)),
            pl.BlockSpec((RB, d_in), lambda i: (i, 0)),
            pl.BlockSpec((RB, 1), lambda i: (i, 0)),
            pl.BlockSpec((1, d_in), lambda i: (0, 0)),
            pl.BlockSpec((d_in, d_out), lambda i: (0, 0)),
        ],
        out_specs=pl.BlockSpec((RB, d_out), lambda i: (i, 0)),
        out_shape=jax.ShapeDtypeStruct((NPAD, d_out), jnp.float32),
    )(s0, s1, hs, dinv, b, w)


def _stage_z(u0, u1, hc, dinv, bc, noise):
    def body(u0_ref, u1_ref, hc_ref, dinv_ref, bc_ref, n_ref, z_ref):
        g = (u0_ref[...] + u1_ref[...] + hc_ref[...]) * dinv_ref[...]
        g = jnp.maximum(g + bc_ref[...], 0.0)
        mean = g[:, :32]
        ls = g[:, 32:64]
        z_ref[...] = n_ref[...] * jnp.exp(ls) + mean

    return pl.pallas_call(
        body,
        grid=(GRID,),
        in_specs=[
            pl.BlockSpec((RB, D_HID), lambda i: (i, 0)),
            pl.BlockSpec((RB, D_HID), lambda i: (i, 0)),
            pl.BlockSpec((RB, D_HID), lambda i: (i, 0)),
            pl.BlockSpec((RB, 1), lambda i: (i, 0)),
            pl.BlockSpec((1, D_HID), lambda i: (0, 0)),
            pl.BlockSpec((RB, 32), lambda i: (i, 0)),
        ],
        out_specs=pl.BlockSpec((RB, 32), lambda i: (i, 0)),
        out_shape=jax.ShapeDtypeStruct((NPAD, 32), jnp.float32),
    )(u0, u1, hc, dinv, bc, noise)


MB, NB = 1000, 4096


def _decoder(z):
    gm = N // MB
    gn = pl.cdiv(N, NB)

    def body(zr_ref, zc_ref, o_ref):
        p = lax.dot_general(zr_ref[...], zc_ref[...],
                            (((1,), (1,)), ((), ())),
                            preferred_element_type=jnp.float32)
        rid = lax.broadcasted_iota(jnp.int32, (MB, NB), 0) + pl.program_id(0) * MB
        cid = lax.broadcasted_iota(jnp.int32, (MB, NB), 1) + pl.program_id(1) * NB
        o_ref[...] = jnp.where(cid > rid, jax.nn.sigmoid(p), 0.0)

    return pl.pallas_call(
        body,
        grid=(gm, gn),
        in_specs=[
            pl.BlockSpec((MB, 32), lambda i, j: (i, 0)),
            pl.BlockSpec((NB, 32), lambda i, j: (j, 0)),
        ],
        out_specs=pl.BlockSpec((MB, NB), lambda i, j: (i, j)),
        out_shape=jax.ShapeDtypeStruct((N, N), jnp.float32),
    )(z, z)


def kernel(x, edge_index, W1, b1, W2, b2, Wm, bm, Ws, bs):
    npad_rows = NPAD - N
    pad_idx = (N + jnp.arange(NT * (EPT_PAD - EPT), dtype=jnp.int32)
               % npad_rows).reshape(NT, EPT_PAD - EPT)
    srcp = jnp.concatenate(
        [edge_index[0].reshape(NT, EPT), pad_idx], axis=1).reshape(NT, CH, B)
    dstp = jnp.concatenate(
        [edge_index[1].reshape(NT, EPT), pad_idx], axis=1).reshape(NT, CH, B)
    xp = jnp.pad(x, ((0, NPAD - N), (0, 0)))
    wc = jnp.pad(jnp.concatenate([Wm, Ws], axis=1), ((0, 0), (0, D_HID - D_C)))
    bc = jnp.pad(jnp.concatenate([bm, bs]), (0, D_HID - D_C)).reshape(1, D_HID)
    noise = jax.random.normal(jax.random.key(42), (N, 32), dtype=jnp.float32)
    noise = jnp.pad(noise, ((0, NPAD - N), (0, 0)))

    deg = _deg_call(dstp).reshape(NC, NPAD)
    d0 = deg[0].reshape(NPAD, 1)
    d1 = deg[1].reshape(NPAD, 1)
    hs1, dinv = _stage_b(xp, d0, d1, W1)

    s = _propagate_call(hs1, srcp, dstp, D_HID)
    hs2 = _stage_mid(s[0], s[1], hs1, dinv, b1.reshape(1, -1), W2, D_HID)
    t = _propagate_call(hs2, srcp, dstp, D_HID)
    hc = _stage_mid(t[0], t[1], hs2, dinv, b2.reshape(1, -1), wc, D_HID)
    u = _propagate_call(hc, srcp, dstp, D_HID)
    z = _stage_z(u[0], u[1], hc, dinv, bc, noise)

    return _decoder(z[:N])


# speedup vs baseline: 2.7831x; 1.0342x over previous
import functools

import jax
import jax.numpy as jnp
from jax import lax
from jax.experimental import pallas as pl
from jax.experimental.pallas import tpu as pltpu
from jax.experimental.pallas import tpu_sc as plsc

N = 10000
NPAD = 10112
D_IN = 128
D_HID = 128
D_C = 64
E = 320000
NC, NS = 2, 16
NT = NC * NS
EPT = E // NT
CH = 80
B = 128
EPT_PAD = CH * B
NBUF = 2
G = 40
RPT = NPAD // NS

_mesh = lambda: plsc.VectorSubcoreMesh(
    core_axis_name="c", subcore_axis_name="s", num_cores=NC, num_subcores=NS)


def _fill_f32(ref, value):
    if len(ref.shape) == 1:
        def body(i, _):
            ref[pl.ds(i * 16, 16)] = jnp.full((16,), value, jnp.float32)
            return 0
        lax.fori_loop(0, ref.shape[0] // 16, body, 0)
    else:
        rows, cols = ref.shape

        def body(i, _):
            r = i // (cols // 16)
            t = i % (cols // 16)
            ref[r, pl.ds(t * 16, 16)] = jnp.full((16,), value, jnp.float32)
            return 0
        lax.fori_loop(0, rows * (cols // 16), body, 0)


def _deg_call(dstp):
    @functools.partial(
        pl.kernel,
        out_type=jax.ShapeDtypeStruct((NC * NPAD,), jnp.float32),
        mesh=_mesh(),
        scratch_types=[
            pltpu.VMEM((CH, B), jnp.int32),
            pltpu.VMEM((640,), jnp.float32),
            pltpu.VMEM((B,), jnp.float32),
            pltpu.VMEM_SHARED((NPAD,), jnp.float32),
        ],
    )
    def deg_kernel(dstp_hbm, out_hbm, didx, zbuf, ones, acc):
        c = lax.axis_index("c")
        s = lax.axis_index("s")
        wid = c * NS + s
        _fill_f32(zbuf, 0.0)
        _fill_f32(ones, 1.0)
        pltpu.sync_copy(zbuf.at[pl.ds(0, RPT)], acc.at[pl.ds(s * RPT, RPT)])
        pltpu.sync_copy(dstp_hbm.at[wid], didx)
        plsc.subcore_barrier()

        def body(j, _):
            pltpu.sync_copy(ones, acc.at[didx.at[j]], add=True)
            return 0
        lax.fori_loop(0, CH, body, 0)
        plsc.subcore_barrier()
        pltpu.sync_copy(acc.at[pl.ds(s * RPT, RPT)], zbuf.at[pl.ds(0, RPT)])
        pltpu.sync_copy(zbuf.at[pl.ds(0, RPT)],
                        out_hbm.at[pl.ds(c * NPAD + s * RPT, RPT)])

    return deg_kernel(dstp)


def _propagate_call(hs, srcp, dstp, d):
    @functools.partial(
        pl.kernel,
        out_type=jax.ShapeDtypeStruct((NC, NPAD, d), jnp.float32),
        mesh=_mesh(),
        scratch_types=[
            pltpu.VMEM((G, B), jnp.int32),
            pltpu.VMEM((G, B), jnp.int32),
            [pltpu.VMEM((B, d), jnp.float32)] * NBUF,
            pltpu.VMEM_SHARED((NPAD, d), jnp.float32),
            [pltpu.SemaphoreType.DMA] * NBUF,
            [pltpu.SemaphoreType.DMA] * NBUF,
        ],
    )
    def prop_kernel(hs_hbm, srcp_hbm, dstp_hbm, out_hbm, sidx, didx, rows,
                    acc, sg, ss):
        c = lax.axis_index("c")
        s = lax.axis_index("s")
        wid = c * NS + s
        _fill_f32(rows[0], 0.0)
        base = s * RPT
        for k in range(RPT // B):
            pltpu.sync_copy(rows[0], acc.at[pl.ds(base + k * B, B)])
        rem = RPT % B
        if rem:
            pltpu.sync_copy(rows[0].at[pl.ds(0, rem)],
                            acc.at[pl.ds(base + (RPT // B) * B, rem)])
        plsc.subcore_barrier()

        def group(grp, _):
            g0 = pl.multiple_of(grp * G, G)
            pltpu.sync_copy(srcp_hbm.at[wid, pl.ds(g0, G)], sidx)
            pltpu.sync_copy(dstp_hbm.at[wid, pl.ds(g0, G)], didx)
            gd = [None] * G
            sd = [None] * G
            for j in range(G):
                b = j % NBUF
                if j >= NBUF:
                    sd[j - NBUF].wait()
                gd[j] = pltpu.async_copy(
                    hs_hbm.at[sidx.at[j]], rows[b], sg[b])
                if j >= 1:
                    bb = (j - 1) % NBUF
                    gd[j - 1].wait()
                    sd[j - 1] = pltpu.async_copy(
                        rows[bb], acc.at[didx.at[j - 1]], ss[bb], add=True)
            bb = (G - 1) % NBUF
            gd[G - 1].wait()
            sd[G - 1] = pltpu.async_copy(
                rows[bb], acc.at[didx.at[G - 1]], ss[bb], add=True)
            for j in range(G - NBUF, G):
                sd[j].wait()
            return 0
        lax.fori_loop(0, CH // G, group, 0)
        plsc.subcore_barrier()
        off = 0
        for i, sz in enumerate((B, B, B, B, RPT - 4 * B)):
            buf = rows[i % NBUF].at[pl.ds(0, sz)]
            pltpu.sync_copy(acc.at[pl.ds(base + off, sz)], buf)
            pltpu.sync_copy(buf, out_hbm.at[c, pl.ds(base + off, sz)])
            off += sz

    return prop_kernel(hs, srcp, dstp)


RB = 632
GRID = NPAD // RB


def _stage_b(x, d0, d1, w1):
    def body(x_ref, d0_ref, d1_ref, w_ref, hs_ref, dinv_ref):
        deg = d0_ref[...] + d1_ref[...] + 1.0
        dinv = lax.rsqrt(deg)
        h = jnp.dot(x_ref[...], w_ref[...], preferred_element_type=jnp.float32)
        hs_ref[...] = h * dinv
        dinv_ref[...] = dinv

    return pl.pallas_call(
        body,
        grid=(GRID,),
        in_specs=[
            pl.BlockSpec((RB, D_IN), lambda i: (i, 0)),
            pl.BlockSpec((RB, 1), lambda i: (i, 0)),
            pl.BlockSpec((RB, 1), lambda i: (i, 0)),
            pl.BlockSpec((D_IN, D_HID), lambda i: (0, 0)),
        ],
        out_specs=[
            pl.BlockSpec((RB, D_HID), lambda i: (i, 0)),
            pl.BlockSpec((RB, 1), lambda i: (i, 0)),
        ],
        out_shape=[
            jax.ShapeDtypeStruct((NPAD, D_HID), jnp.float32),
            jax.ShapeDtypeStruct((NPAD, 1), jnp.float32),
        ],
    )(x, d0, d1, w1)


def _stage_mid(s0, s1, hs, dinv, b, w, d_out):
    d_in = hs.shape[1]

    def body(s0_ref, s1_ref, hs_ref, dinv_ref, b_ref, w_ref, o_ref):
        dinv = dinv_ref[...]
        pre = (s0_ref[...] + s1_ref[...] + hs_ref[...]) * dinv + b_ref[...]
        h = jnp.maximum(pre, 0.0)
        o_ref[...] = jnp.dot(h, w_ref[...],
                             preferred_element_type=jnp.float32) * dinv

    return pl.pallas_call(
        body,
        grid=(GRID,),
        in_specs=[
            pl.BlockSpec((RB, d_in), lambda i: (i, 0)),
            pl.BlockSpec((RB, d_in), lambda i: (i, 0)),
            pl.BlockSpec((RB, d_in), lambda i: (i, 0)),
            pl.BlockSpec((RB, 1), lambda i: (i, 0)),
            pl.BlockSpec((1, d_in), lambda i: (0, 0)),
            pl.BlockSpec((d_in, d_out), lambda i: (0, 0)),
        ],
        out_specs=pl.BlockSpec((RB, d_out), lambda i: (i, 0)),
        out_shape=jax.ShapeDtypeStruct((NPAD, d_out), jnp.float32),
    )(s0, s1, hs, dinv, b, w)


def _stage_z(u0, u1, hc, dinv, bc, noise):
    def body(u0_ref, u1_ref, hc_ref, dinv_ref, bc_ref, n_ref, z_ref):
        g = (u0_ref[...] + u1_ref[...] + hc_ref[...]) * dinv_ref[...]
        g = jnp.maximum(g + bc_ref[...], 0.0)
        mean = g[:, :32]
        ls = g[:, 32:64]
        z_ref[...] = n_ref[...] * jnp.exp(ls) + mean

    return pl.pallas_call(
        body,
        grid=(GRID,),
        in_specs=[
            pl.BlockSpec((RB, D_HID), lambda i: (i, 0)),
            pl.BlockSpec((RB, D_HID), lambda i: (i, 0)),
            pl.BlockSpec((RB, D_HID), lambda i: (i, 0)),
            pl.BlockSpec((RB, 1), lambda i: (i, 0)),
            pl.BlockSpec((1, D_HID), lambda i: (0, 0)),
            pl.BlockSpec((RB, 32), lambda i: (i, 0)),
        ],
        out_specs=pl.BlockSpec((RB, 32), lambda i: (i, 0)),
        out_shape=jax.ShapeDtypeStruct((NPAD, 32), jnp.float32),
    )(u0, u1, hc, dinv, bc, noise)


MB, NB = 1000, 5120


def _decoder(z):
    gm = N // MB
    gn = pl.cdiv(N, NB)

    def body(zr_ref, zc_ref, o_ref):
        p = lax.dot_general(zr_ref[...], zc_ref[...],
                            (((1,), (1,)), ((), ())),
                            preferred_element_type=jnp.float32)
        rid = lax.broadcasted_iota(jnp.int32, (MB, NB), 0) + pl.program_id(0) * MB
        cid = lax.broadcasted_iota(jnp.int32, (MB, NB), 1) + pl.program_id(1) * NB
        o_ref[...] = jnp.where(cid > rid, jax.nn.sigmoid(p), 0.0)

    return pl.pallas_call(
        body,
        grid=(gm, gn),
        in_specs=[
            pl.BlockSpec((MB, 32), lambda i, j: (i, 0)),
            pl.BlockSpec((NB, 32), lambda i, j: (j, 0)),
        ],
        out_specs=pl.BlockSpec((MB, NB), lambda i, j: (i, j)),
        out_shape=jax.ShapeDtypeStruct((N, N), jnp.float32),
    )(z, z)


def kernel(x, edge_index, W1, b1, W2, b2, Wm, bm, Ws, bs):
    npad_rows = NPAD - N
    pad_idx = (N + jnp.arange(NT * (EPT_PAD - EPT), dtype=jnp.int32)
               % npad_rows).reshape(NT, EPT_PAD - EPT)
    srcp = jnp.concatenate(
        [edge_index[0].reshape(NT, EPT), pad_idx], axis=1).reshape(NT, CH, B)
    dstp = jnp.concatenate(
        [edge_index[1].reshape(NT, EPT), pad_idx], axis=1).reshape(NT, CH, B)
    xp = jnp.pad(x, ((0, NPAD - N), (0, 0)))
    wc = jnp.pad(jnp.concatenate([Wm, Ws], axis=1), ((0, 0), (0, D_HID - D_C)))
    bc = jnp.pad(jnp.concatenate([bm, bs]), (0, D_HID - D_C)).reshape(1, D_HID)
    noise = jax.random.normal(jax.random.key(42), (N, 32), dtype=jnp.float32)
    noise = jnp.pad(noise, ((0, NPAD - N), (0, 0)))

    deg = _deg_call(dstp).reshape(NC, NPAD)
    d0 = deg[0].reshape(NPAD, 1)
    d1 = deg[1].reshape(NPAD, 1)
    hs1, dinv = _stage_b(xp, d0, d1, W1)

    s = _propagate_call(hs1, srcp, dstp, D_HID)
    hs2 = _stage_mid(s[0], s[1], hs1, dinv, b1.reshape(1, -1), W2, D_HID)
    t = _propagate_call(hs2, srcp, dstp, D_HID)
    hc = _stage_mid(t[0], t[1], hs2, dinv, b2.reshape(1, -1), wc, D_HID)
    u = _propagate_call(hc, srcp, dstp, D_HID)
    z = _stage_z(u[0], u[1], hc, dinv, bc, noise)

    return _decoder(z[:N])
